# baseline (speedup 1.0000x reference)
import functools

import jax
import jax.numpy as jnp
from jax import lax
from jax.experimental import pallas as pl
from jax.experimental.pallas import tpu as pltpu
from jax.experimental.pallas import tpu_sc as plsc

_N = 4096
_TC_ROWS = 2048
_SC_ROWS = _N - _TC_ROWS
_ROW_BLK = 512
_NW = 32
_RPW = _SC_ROWS // _NW
_CHUNK = 4
_GROUP = 16
_L = 16
_NCH = _N // _L


def _tc_kernel(x_ref, y_ref, c_ref, q_ref, at_ref, top_ref, bot_ref):
    i = pl.program_id(0)
    z = (
        jax.lax.dot(q_ref[...], x_ref[...], preferred_element_type=jnp.float32)
        + jax.lax.dot(at_ref[...], y_ref[...], preferred_element_type=jnp.float32)
        + c_ref[pl.ds(i * _ROW_BLK, _ROW_BLK), :]
    )
    partial_top = jnp.sum(jnp.abs(z))

    @pl.when(i == 0)
    def _init():
        top_ref[0, 0] = partial_top
        bot_ref[0, 0] = jnp.sum(jnp.abs(c_ref[...]))

    @pl.when(i != 0)
    def _acc():
        top_ref[0, 0] += partial_top


def _tc_call(Q, AT, c2, x, y):
    grid = (_TC_ROWS // _ROW_BLK,)
    return pl.pallas_call(
        _tc_kernel,
        grid=grid,
        in_specs=[
            pl.BlockSpec((_N, 1), lambda i: (0, 0)),
            pl.BlockSpec((_N, 1), lambda i: (0, 0)),
            pl.BlockSpec((_N, 1), lambda i: (0, 0)),
            pl.BlockSpec((_ROW_BLK, _N), lambda i: (i, 0)),
            pl.BlockSpec((_ROW_BLK, _N), lambda i: (i, 0)),
        ],
        out_specs=[
            pl.BlockSpec((1, 1), lambda i: (0, 0), memory_space=pltpu.SMEM),
            pl.BlockSpec((1, 1), lambda i: (0, 0), memory_space=pltpu.SMEM),
        ],
        out_shape=[
            jax.ShapeDtypeStruct((1, 1), jnp.float32),
            jax.ShapeDtypeStruct((1, 1), jnp.float32),
        ],
    )(x, y, c2, Q, AT)


def _sc_body(q_hbm, at_hbm, c_hbm, x_hbm, y_hbm, out_hbm,
             x_v, y_v, c_v, qb0, qb1, ab0, ab1, o_v,
             sq0, sq1, sa0, sa1):
    wid = lax.axis_index("s") * 2 + lax.axis_index("c")
    base = _TC_ROWS + wid * _RPW

    pltpu.sync_copy(x_hbm, x_v)
    pltpu.sync_copy(y_hbm, y_v)
    pltpu.sync_copy(c_hbm.at[pl.ds(base, _RPW)], c_v)

    qbufs = (qb0, qb1)
    abufs = (ab0, ab1)
    qsems = (sq0, sq1)
    asems = (sa0, sa1)

    def start(t):
        e0 = (base + t * _CHUNK) * _N
        k = t % 2
        cq = pltpu.async_copy(q_hbm.at[pl.ds(e0, _CHUNK * _N)], qbufs[k], qsems[k])
        ca = pltpu.async_copy(at_hbm.at[pl.ds(e0, _CHUNK * _N)], abufs[k], asems[k])
        return cq, ca

    n_chunks = _RPW // _CHUNK
    pending = start(0)
    pacc = jnp.zeros((_L,), jnp.float32)
    lane = lax.iota(jnp.int32, _L)
    zero = jnp.zeros((_L,), jnp.float32)
    zvec = zero

    for t in range(n_chunks):
        k = t % 2
        qb = qbufs[k]
        ab = abufs[k]
        nxt = start(t + 1) if t + 1 < n_chunks else None
        pending[0].wait()
        pending[1].wait()
        pending = nxt

        def body(j, accs, qb=qb, ab=ab):
            col = j * _L
            xv = x_v[pl.ds(col, _L)]
            yv = y_v[pl.ds(col, _L)]
            out = []
            for r in range(_CHUNK):
                out.append(accs[r] + qb[pl.ds(r * _N + col, _L)] * xv
                           + ab[pl.ds(r * _N + col, _L)] * yv)
            return tuple(out)

        accs = lax.fori_loop(0, _NCH, body, (zero,) * _CHUNK, unroll=8)
        for r in range(_CHUNK):
            p = (t % (_GROUP // _CHUNK)) * _CHUNK + r
            s = jnp.sum(accs[r])
            zvec = jnp.where(lane == p, s, zvec)

        if t % (_GROUP // _CHUNK) == _GROUP // _CHUNK - 1:
            g = t // (_GROUP // _CHUNK)
            z16 = zvec + c_v[pl.ds(g * _GROUP, _GROUP)]
            pacc = pacc + jnp.abs(z16)
            zvec = zero

    o_v[...] = pacc
    pltpu.sync_copy(o_v, out_hbm.at[wid])


def _sc_call(Q, AT, c, x1, y1):
    mesh = plsc.VectorSubcoreMesh(core_axis_name="c", subcore_axis_name="s")
    fn = pl.kernel(
        _sc_body,
        out_type=jax.ShapeDtypeStruct((_NW, _L), jnp.float32),
        mesh=mesh,
        scratch_types=[
            pltpu.VMEM((_N,), jnp.float32),
            pltpu.VMEM((_N,), jnp.float32),
            pltpu.VMEM((_RPW,), jnp.float32),
            pltpu.VMEM((_CHUNK * _N,), jnp.float32),
            pltpu.VMEM((_CHUNK * _N,), jnp.float32),
            pltpu.VMEM((_CHUNK * _N,), jnp.float32),
            pltpu.VMEM((_CHUNK * _N,), jnp.float32),
            pltpu.VMEM((_L,), jnp.float32),
            pltpu.SemaphoreType.DMA,
            pltpu.SemaphoreType.DMA,
            pltpu.SemaphoreType.DMA,
            pltpu.SemaphoreType.DMA,
        ],
        compiler_params=pltpu.CompilerParams(needs_layout_passes=False),
    )
    return fn(Q, AT, c, x1, y1)


def kernel(Q, AT, b, c, x, y):
    del b
    c2 = c[:, None]
    top_tc, bot = _tc_call(Q, AT, c2, x, y)
    sc_part = _sc_call(Q.reshape(-1), AT.reshape(-1), c, x[:, 0], y[:, 0])
    top = top_tc[0, 0] + jnp.sum(sc_part)
    return top / (10000.0 + bot[0, 0])


# speedup vs baseline: 2.3101x; 2.3101x over previous
import functools

import jax
import jax.numpy as jnp
from jax import lax
from jax.experimental import pallas as pl
from jax.experimental.pallas import tpu as pltpu
from jax.experimental.pallas import tpu_sc as plsc

_N = 4096
_TC_ROWS = 2048
_SC_ROWS = _N - _TC_ROWS
_ROW_BLK = 512
_NW = 32
_RPW = _SC_ROWS // _NW
_CHUNK = 2
_NBUF = 4
_GROUP = 16
_L = 16
_NCH = _N // _L


def _tc_kernel(x_ref, y_ref, c_ref, q_ref, at_ref, top_ref, bot_ref):
    i = pl.program_id(0)
    z = (
        jax.lax.dot(q_ref[...], x_ref[...], preferred_element_type=jnp.float32)
        + jax.lax.dot(at_ref[...], y_ref[...], preferred_element_type=jnp.float32)
        + c_ref[pl.ds(i * _ROW_BLK, _ROW_BLK), :]
    )
    partial_top = jnp.sum(jnp.abs(z))

    @pl.when(i == 0)
    def _init():
        top_ref[0, 0] = partial_top
        bot_ref[0, 0] = jnp.sum(jnp.abs(c_ref[...]))

    @pl.when(i != 0)
    def _acc():
        top_ref[0, 0] += partial_top


def _tc_call(Q, AT, c2, x, y):
    grid = (_TC_ROWS // _ROW_BLK,)
    return pl.pallas_call(
        _tc_kernel,
        grid=grid,
        in_specs=[
            pl.BlockSpec((_N, 1), lambda i: (0, 0)),
            pl.BlockSpec((_N, 1), lambda i: (0, 0)),
            pl.BlockSpec((_N, 1), lambda i: (0, 0)),
            pl.BlockSpec((_ROW_BLK, _N), lambda i: (i, 0)),
            pl.BlockSpec((_ROW_BLK, _N), lambda i: (i, 0)),
        ],
        out_specs=[
            pl.BlockSpec((1, 1), lambda i: (0, 0), memory_space=pltpu.SMEM),
            pl.BlockSpec((1, 1), lambda i: (0, 0), memory_space=pltpu.SMEM),
        ],
        out_shape=[
            jax.ShapeDtypeStruct((1, 1), jnp.float32),
            jax.ShapeDtypeStruct((1, 1), jnp.float32),
        ],
    )(x, y, c2, Q, AT)


def _sc_body(q_hbm, at_hbm, c_hbm, x_hbm, y_hbm, out_hbm,
             x_v, y_v, c_v, o_v,
             qb0, qb1, qb2, qb3, ab0, ab1, ab2, ab3,
             sq0, sq1, sq2, sq3, sa0, sa1, sa2, sa3):
    wid = lax.axis_index("s") * 2 + lax.axis_index("c")
    base = _TC_ROWS + wid * _RPW

    pltpu.sync_copy(x_hbm, x_v)
    pltpu.sync_copy(y_hbm, y_v)
    pltpu.sync_copy(c_hbm.at[pl.ds(base, _RPW)], c_v)

    qbufs = (qb0, qb1, qb2, qb3)
    abufs = (ab0, ab1, ab2, ab3)
    qsems = (sq0, sq1, sq2, sq3)
    asems = (sa0, sa1, sa2, sa3)

    def start(t):
        r0 = base + t * _CHUNK
        k = t % _NBUF
        cq = pltpu.async_copy(q_hbm.at[pl.ds(r0, _CHUNK), :], qbufs[k], qsems[k])
        ca = pltpu.async_copy(at_hbm.at[pl.ds(r0, _CHUNK), :], abufs[k], asems[k])
        return cq, ca

    n_chunks = _RPW // _CHUNK
    pending = [start(t) for t in range(_NBUF - 1)]
    pacc = jnp.zeros((_L,), jnp.float32)
    lane = lax.iota(jnp.int32, _L)
    zero = jnp.zeros((_L,), jnp.float32)
    zvec = zero

    for t in range(n_chunks):
        k = t % _NBUF
        qb = qbufs[k]
        ab = abufs[k]
        if t + _NBUF - 1 < n_chunks:
            pending.append(start(t + _NBUF - 1))
        cur = pending.pop(0)
        cur[0].wait()
        cur[1].wait()

        def body(j, accs, qb=qb, ab=ab):
            col = j * _L
            xv = x_v[pl.ds(col, _L)]
            yv = y_v[pl.ds(col, _L)]
            out = []
            for r in range(_CHUNK):
                out.append(accs[r] + qb[r, pl.ds(col, _L)] * xv
                           + ab[r, pl.ds(col, _L)] * yv)
            return tuple(out)

        accs = lax.fori_loop(0, _NCH, body, (zero,) * _CHUNK, unroll=4)
        for r in range(_CHUNK):
            p = (t % (_GROUP // _CHUNK)) * _CHUNK + r
            s = jnp.sum(accs[r])
            zvec = jnp.where(lane == p, s, zvec)

        if t % (_GROUP // _CHUNK) == _GROUP // _CHUNK - 1:
            g = t // (_GROUP // _CHUNK)
            z16 = zvec + c_v[pl.ds(g * _GROUP, _GROUP)]
            pacc = pacc + jnp.abs(z16)
            zvec = zero

    o_v[...] = pacc
    pltpu.sync_copy(o_v, out_hbm.at[wid])


def _sc_call(Q, AT, c, x1, y1):
    mesh = plsc.VectorSubcoreMesh(core_axis_name="c", subcore_axis_name="s")
    fn = pl.kernel(
        _sc_body,
        out_type=jax.ShapeDtypeStruct((_NW, _L), jnp.float32),
        mesh=mesh,
        scratch_types=[
            pltpu.VMEM((_N,), jnp.float32),
            pltpu.VMEM((_N,), jnp.float32),
            pltpu.VMEM((_RPW,), jnp.float32),
            pltpu.VMEM((_L,), jnp.float32),
        ] + [pltpu.VMEM((_CHUNK, _N), jnp.float32)] * (2 * _NBUF)
          + [pltpu.SemaphoreType.DMA] * (2 * _NBUF),
        compiler_params=pltpu.CompilerParams(needs_layout_passes=False),
    )
    return fn(Q, AT, c, x1, y1)


def kernel(Q, AT, b, c, x, y):
    del b
    c2 = c[:, None]
    top_tc, bot = _tc_call(Q, AT, c2, x, y)
    sc_part = _sc_call(Q, AT, c, x[:, 0], y[:, 0])
    top = top_tc[0, 0] + jnp.sum(sc_part)
    return top / (10000.0 + bot[0, 0])


# speedup vs baseline: 3.3143x; 1.4347x over previous
import jax
import jax.numpy as jnp
from jax import lax
from jax.experimental import pallas as pl
from jax.experimental.pallas import tpu as pltpu

_N = 4096
_ROW_BLK = 256
_RING = 4
_NSTEP = _N // _ROW_BLK


def _tc_kernel(x_ref, y_ref, c_ref, q_hbm, at_hbm, top_ref, bot_ref,
               buf, zscr, sem):
    i = pl.program_id(0)

    def fetch(step, k):
        @pl.when(step < _NSTEP)
        def _q():
            pltpu.make_async_copy(
                q_hbm.at[pl.ds(step * _ROW_BLK, _ROW_BLK), :],
                buf.at[k], sem.at[k]
            ).start()

        @pl.when(jnp.logical_and(step >= _NSTEP, step < 2 * _NSTEP))
        def _at():
            pltpu.make_async_copy(
                at_hbm.at[pl.ds((step - _NSTEP) * _ROW_BLK, _ROW_BLK), :],
                buf.at[k], sem.at[k]
            ).start()

    @pl.when(i == 0)
    def _prologue():
        for k in range(_RING - 1):
            fetch(jnp.int32(k), jnp.int32(k))

    fetch(i + _RING - 1, lax.rem(i + _RING - 1, _RING))

    k = lax.rem(i, _RING)
    pltpu.make_async_copy(
        q_hbm.at[pl.ds(0, _ROW_BLK), :], buf.at[k], sem.at[k]
    ).wait()

    r0 = lax.rem(i, _NSTEP) * _ROW_BLK

    @pl.when(i < _NSTEP)
    def _phase_q():
        zscr[pl.ds(r0, _ROW_BLK), :] = jax.lax.dot(
            buf[k], x_ref[...], preferred_element_type=jnp.float32)

    @pl.when(i >= _NSTEP)
    def _phase_at():
        z = (
            jax.lax.dot(buf[k], y_ref[...], preferred_element_type=jnp.float32)
            + zscr[pl.ds(r0, _ROW_BLK), :]
            + c_ref[pl.ds(r0, _ROW_BLK), :]
        )
        partial_top = jnp.sum(jnp.abs(z))

        @pl.when(i == _NSTEP)
        def _init():
            top_ref[0, 0] = partial_top
            bot_ref[0, 0] = jnp.sum(jnp.abs(c_ref[...]))

        @pl.when(i != _NSTEP)
        def _acc():
            top_ref[0, 0] += partial_top


def kernel(Q, AT, b, c, x, y):
    del b
    c2 = c[:, None]
    top, bot = pl.pallas_call(
        _tc_kernel,
        grid=(2 * _NSTEP,),
        in_specs=[
            pl.BlockSpec((_N, 1), lambda i: (0, 0)),
            pl.BlockSpec((_N, 1), lambda i: (0, 0)),
            pl.BlockSpec((_N, 1), lambda i: (0, 0)),
            pl.BlockSpec(memory_space=pltpu.HBM),
            pl.BlockSpec(memory_space=pltpu.HBM),
        ],
        out_specs=[
            pl.BlockSpec((1, 1), lambda i: (0, 0), memory_space=pltpu.SMEM),
            pl.BlockSpec((1, 1), lambda i: (0, 0), memory_space=pltpu.SMEM),
        ],
        out_shape=[
            jax.ShapeDtypeStruct((1, 1), jnp.float32),
            jax.ShapeDtypeStruct((1, 1), jnp.float32),
        ],
        scratch_shapes=[
            pltpu.VMEM((_RING, _ROW_BLK, _N), jnp.float32),
            pltpu.VMEM((_N, 1), jnp.float32),
            pltpu.SemaphoreType.DMA((_RING,)),
        ],
        compiler_params=pltpu.CompilerParams(
            dimension_semantics=("arbitrary",)),
    )(x, y, c2, Q, AT)
    return top[0, 0] / (10000.0 + bot[0, 0])
